# Initial kernel scaffold; baseline (speedup 1.0000x reference)
import jax
import jax.numpy as jnp
from jax.experimental import pallas as pl


def kernel(origins, directions, lengths, density_grid, color_grid):
    raise NotImplementedError("write your pallas kernel here")


# speedup vs baseline: 3.1422x; 3.1422x over previous
import functools

import jax
import jax.numpy as jnp
import numpy as np
from jax import lax
from jax.experimental import pallas as pl
from jax.experimental.pallas import tpu as pltpu
from jax.experimental.pallas import tpu_sc as plsc

_ALPHA_INIT = 1e-06
_ACT_SHIFT = float(np.log(1.0 / (1.0 - _ALPHA_INIT) - 1.0))

_N_RAYS = 4096
_N_PTS = 128
_G = 100
_NC = 2
_NS = 16
_NW = _NC * _NS
_RAYS_PER_W = _N_RAYS // _NW
_L = 16
_NB = _N_PTS // _L


def _sc_body(origins, directions, lengths, tab,
             rawd, rawc,
             o_v, d_v, len_v, idx_v, w_v, co_v, dst_v, dout_v, cout_v, sem):
    wid = lax.axis_index("s") * _NC + lax.axis_index("c")
    r0 = wid * _RAYS_PER_W

    pltpu.sync_copy(origins.at[pl.ds(r0 * 3, _RAYS_PER_W * 3)], o_v)
    pltpu.sync_copy(directions.at[pl.ds(r0 * 3, _RAYS_PER_W * 3)], d_v)
    pltpu.sync_copy(lengths.at[pl.ds(r0 * _N_PTS, _RAYS_PER_W * _N_PTS)], len_v)

    iota = lax.iota(jnp.int32, _L)
    one = jnp.ones((_L,), jnp.float32)
    zero = jnp.zeros((_L,), jnp.float32)

    def ray_body(r, carry):
        rv3 = jnp.full((_L,), 3 * r, jnp.int32)
        ox = plsc.load_gather(o_v, [rv3])
        oy = plsc.load_gather(o_v, [rv3 + 1])
        oz = plsc.load_gather(o_v, [rv3 + 2])
        dx = plsc.load_gather(d_v, [rv3])
        dy = plsc.load_gather(d_v, [rv3 + 1])
        dz = plsc.load_gather(d_v, [rv3 + 2])
        lbase = r * _N_PTS

        for b in range(_NB):
            pidx = b * _L + iota
            t = len_v[pl.ds(lbase + b * _L, _L)]

            def axis_prep(o_c, d_c):
                f = (o_c + d_c * t + 1.0) * 0.5 * (_G - 1)
                i0 = f.astype(jnp.int32)
                i0 = i0 - jnp.where(i0.astype(jnp.float32) > f, 1, 0)
                frac = f - i0.astype(jnp.float32)
                i1 = i0 + 1
                v0 = jnp.where((i0 >= 0) & (i0 <= _G - 1), one, zero)
                v1 = jnp.where((i1 >= 0) & (i1 <= _G - 1), one, zero)
                w0 = (1.0 - frac) * v0
                w1 = frac * v1
                return i0, i1, w0, w1

            x0, x1, wx0, wx1 = axis_prep(ox, dx)
            y0, y1, wy0, wy1 = axis_prep(oy, dy)
            z0, z1, wz0, wz1 = axis_prep(oz, dz)

            xr = jnp.clip(x0, 0, _G - 1)
            yc = (jnp.clip(y0, 0, _G - 1), jnp.clip(y1, 0, _G - 1))
            zc = (jnp.clip(z0, 0, _G - 1), jnp.clip(z1, 0, _G - 1))
            wys = (wy0, wy1)
            wzs = (wz0, wz1)
            coff = jnp.where(x0 >= 0, 4, 0)
            co_v[pl.ds(b * _L, _L)] = coff
            w_v[pl.ds(4 * _N_PTS + b * _L, _L)] = wx0
            w_v[pl.ds(5 * _N_PTS + b * _L, _L)] = wx1

            for kz in range(2):
                zbase = zc[kz] * (_G * _G)
                for ky in range(2):
                    k = kz * 2 + ky
                    kv = jnp.full((_L,), k, jnp.int32)
                    cidx = zbase + yc[ky] * _G + xr
                    plsc.store_scatter(idx_v, [kv, pidx], cidx)
                    w_v[pl.ds(k * _N_PTS + b * _L, _L)] = wzs[kz] * wys[ky]

        descs = [pltpu.async_copy(tab.at[idx_v.at[k]], dst_v.at[k], sem)
                 for k in range(4)]
        for dsc in descs:
            dsc.wait()

        for b in range(_NB):
            pidx = b * _L + iota
            wx0 = w_v[pl.ds(4 * _N_PTS + b * _L, _L)]
            wx1 = w_v[pl.ds(5 * _N_PTS + b * _L, _L)]
            coff = co_v[pl.ds(b * _L, _L)]
            acc = [zero, zero, zero, zero]
            for k in range(4):
                kv = jnp.full((_L,), k, jnp.int32)
                wzy = w_v[pl.ds(k * _N_PTS + b * _L, _L)]
                wa = wzy * wx0
                wb = wzy * wx1
                for c in range(4):
                    cv = jnp.full((_L,), c, jnp.int32)
                    v0 = plsc.load_gather(dst_v, [kv, pidx, cv])
                    v1 = plsc.load_gather(dst_v, [kv, pidx, cv + coff])
                    acc[c] = acc[c] + wa * v0 + wb * v1
            dout_v[pl.ds(b * _L, _L)] = acc[0]
            cbase = pidx * 3
            plsc.store_scatter(cout_v, [cbase], acc[1])
            plsc.store_scatter(cout_v, [cbase + 1], acc[2])
            plsc.store_scatter(cout_v, [cbase + 2], acc[3])

        pltpu.sync_copy(dout_v, rawd.at[r0 + r])
        pltpu.sync_copy(cout_v, rawc.at[r0 + r])
        return carry

    lax.fori_loop(0, _RAYS_PER_W, ray_body, 0)


@jax.jit
def _sc_sample(origins_f, directions_f, lengths_f, tab):
    mesh = plsc.VectorSubcoreMesh(core_axis_name="c", subcore_axis_name="s")
    fn = functools.partial(
        pl.kernel,
        out_type=(
            jax.ShapeDtypeStruct((_N_RAYS, _N_PTS), jnp.float32),
            jax.ShapeDtypeStruct((_N_RAYS, 3 * _N_PTS), jnp.float32),
        ),
        mesh=mesh,
        compiler_params=pltpu.CompilerParams(
            needs_layout_passes=False, use_tc_tiling_on_sc=False),
        scratch_types=[
            pltpu.VMEM((_RAYS_PER_W * 3,), jnp.float32),
            pltpu.VMEM((_RAYS_PER_W * 3,), jnp.float32),
            pltpu.VMEM((_RAYS_PER_W * _N_PTS,), jnp.float32),
            pltpu.VMEM((4, _N_PTS), jnp.int32),
            pltpu.VMEM((6 * _N_PTS,), jnp.float32),
            pltpu.VMEM((_N_PTS,), jnp.int32),
            pltpu.VMEM((4, _N_PTS, 8), jnp.float32),
            pltpu.VMEM((_N_PTS,), jnp.float32),
            pltpu.VMEM((3 * _N_PTS,), jnp.float32),
            pltpu.SemaphoreType.DMA,
        ],
    )(_sc_body)
    return fn(origins_f, directions_f, lengths_f, tab)


def _act_body(dir_ref, rawd_ref, rawc_ref, dens_ref, col_ref):
    dvec = dir_ref[...]
    interval = jnp.sqrt(jnp.sum(dvec * dvec, axis=1, keepdims=True))
    a = rawd_ref[...] + _ACT_SHIFT
    dens_ref[...] = 1.0 - (1.0 + jnp.exp(a)) ** (-interval)
    col_ref[...] = jax.nn.sigmoid(rawc_ref[...])


@jax.jit
def _activations(directions, rawd, rawc):
    blk = 1024
    grid = (_N_RAYS // blk,)
    return pl.pallas_call(
        _act_body,
        grid=grid,
        in_specs=[
            pl.BlockSpec((blk, 3), lambda i: (i, 0)),
            pl.BlockSpec((blk, _N_PTS), lambda i: (i, 0)),
            pl.BlockSpec((blk, 3 * _N_PTS), lambda i: (i, 0)),
        ],
        out_specs=[
            pl.BlockSpec((blk, _N_PTS), lambda i: (i, 0)),
            pl.BlockSpec((blk, 3 * _N_PTS), lambda i: (i, 0)),
        ],
        out_shape=[
            jax.ShapeDtypeStruct((_N_RAYS, _N_PTS), jnp.float32),
            jax.ShapeDtypeStruct((_N_RAYS, 3 * _N_PTS), jnp.float32),
        ],
    )(directions, rawd, rawc)


def kernel(origins, directions, lengths, density_grid, color_grid):
    fused = jnp.stack(
        [density_grid[0, 0], color_grid[0, 0], color_grid[0, 1], color_grid[0, 2]],
        axis=-1,
    ).reshape(_G * _G * _G, 4)
    nxt = jnp.concatenate([fused[1:], jnp.zeros((1, 4), jnp.float32)], axis=0)
    tab = jnp.concatenate([fused, nxt], axis=1)
    rawd, rawc = _sc_sample(
        origins.reshape(-1), directions.reshape(-1), lengths.reshape(-1), tab)
    dens, col = _activations(directions, rawd, rawc)
    return (dens.reshape(_N_RAYS, _N_PTS, 1), col.reshape(_N_RAYS, _N_PTS, 3))


# speedup vs baseline: 4.3731x; 1.3918x over previous
import functools

import jax
import jax.numpy as jnp
import numpy as np
from jax import lax
from jax.experimental import pallas as pl
from jax.experimental.pallas import tpu as pltpu
from jax.experimental.pallas import tpu_sc as plsc

_ALPHA_INIT = 1e-06
_ACT_SHIFT = float(np.log(1.0 / (1.0 - _ALPHA_INIT) - 1.0))

_N_RAYS = 4096
_N_PTS = 128
_G = 100
_NC = 2
_NS = 16
_NW = _NC * _NS
_RAYS_PER_W = _N_RAYS // _NW
_L = 16
_NB = _N_PTS // _L


_NV = _G * _G * _G
_CH_ROWS = 2000
_N_CHUNKS = _NV // _CH_ROWS
_CH_IN = _CH_ROWS + 16


def _sc_body(origins, directions, lengths, density_f, color_f,
             rawd, rawc, tab,
             o_v, d_v, len_v, idx_v, w_v, co_v, dst_v, dout_v, cout_v,
             din, c0in, c1in, c2in, tout_v, sem):
    wid = lax.axis_index("s") * _NC + lax.axis_index("c")
    sid = lax.axis_index("s")
    r0 = wid * _RAYS_PER_W

    iota = lax.iota(jnp.int32, _L)
    one = jnp.ones((_L,), jnp.float32)
    zero = jnp.zeros((_L,), jnp.float32)

    def chunk_body(i, carry):
        j = sid + i * _NS

        @pl.when(j < _N_CHUNKS)
        def _():
            v0 = j * _CH_ROWS
            s = jnp.minimum(v0, _NV - _CH_IN)
            off = v0 - s
            pltpu.sync_copy(density_f.at[pl.ds(s, _CH_IN)], din)
            pltpu.sync_copy(color_f.at[pl.ds(s, _CH_IN)], c0in)
            pltpu.sync_copy(color_f.at[pl.ds(_NV + s, _CH_IN)], c1in)
            pltpu.sync_copy(color_f.at[pl.ds(2 * _NV + s, _CH_IN)], c2in)
            for it in range(_CH_ROWS // _L):
                base = off + it * _L
                ridx = it * _L + iota
                gv1 = v0 + it * _L + iota + 1
                ok = gv1 <= _NV - 1
                for c, ref in enumerate((din, c0in, c1in, c2in)):
                    cv = jnp.full((_L,), c, jnp.int32)
                    plsc.store_scatter(tout_v, [ridx, cv], ref[pl.ds(base, _L)])
                    sh = jnp.where(ok, ref[pl.ds(base + 1, _L)], zero)
                    plsc.store_scatter(tout_v, [ridx, cv + 4], sh)
            pltpu.sync_copy(tout_v, tab.at[pl.ds(v0, _CH_ROWS)])

        return carry

    lax.fori_loop(0, (_N_CHUNKS + _NS - 1) // _NS, chunk_body, 0)
    plsc.subcore_barrier()

    pltpu.sync_copy(origins.at[pl.ds(r0 * 3, _RAYS_PER_W * 3)], o_v)
    pltpu.sync_copy(directions.at[pl.ds(r0 * 3, _RAYS_PER_W * 3)], d_v)
    pltpu.sync_copy(lengths.at[pl.ds(r0 * _N_PTS, _RAYS_PER_W * _N_PTS)], len_v)

    def ray_body(r, carry):
        rv3 = jnp.full((_L,), 3 * r, jnp.int32)
        ox = plsc.load_gather(o_v, [rv3])
        oy = plsc.load_gather(o_v, [rv3 + 1])
        oz = plsc.load_gather(o_v, [rv3 + 2])
        dx = plsc.load_gather(d_v, [rv3])
        dy = plsc.load_gather(d_v, [rv3 + 1])
        dz = plsc.load_gather(d_v, [rv3 + 2])
        lbase = r * _N_PTS

        for b in range(_NB):
            pidx = b * _L + iota
            t = len_v[pl.ds(lbase + b * _L, _L)]

            def axis_prep(o_c, d_c):
                f = (o_c + d_c * t + 1.0) * 0.5 * (_G - 1)
                i0 = f.astype(jnp.int32)
                i0 = i0 - jnp.where(i0.astype(jnp.float32) > f, 1, 0)
                frac = f - i0.astype(jnp.float32)
                i1 = i0 + 1
                v0 = jnp.where((i0 >= 0) & (i0 <= _G - 1), one, zero)
                v1 = jnp.where((i1 >= 0) & (i1 <= _G - 1), one, zero)
                w0 = (1.0 - frac) * v0
                w1 = frac * v1
                return i0, i1, w0, w1

            x0, x1, wx0, wx1 = axis_prep(ox, dx)
            y0, y1, wy0, wy1 = axis_prep(oy, dy)
            z0, z1, wz0, wz1 = axis_prep(oz, dz)

            xr = jnp.clip(x0, 0, _G - 1)
            yc = (jnp.clip(y0, 0, _G - 1), jnp.clip(y1, 0, _G - 1))
            zc = (jnp.clip(z0, 0, _G - 1), jnp.clip(z1, 0, _G - 1))
            wys = (wy0, wy1)
            wzs = (wz0, wz1)
            coff = jnp.where(x0 >= 0, 4, 0)
            co_v[pl.ds(b * _L, _L)] = coff
            w_v[pl.ds(4 * _N_PTS + b * _L, _L)] = wx0
            w_v[pl.ds(5 * _N_PTS + b * _L, _L)] = wx1

            for kz in range(2):
                zbase = zc[kz] * (_G * _G)
                for ky in range(2):
                    k = kz * 2 + ky
                    kv = jnp.full((_L,), k, jnp.int32)
                    cidx = zbase + yc[ky] * _G + xr
                    plsc.store_scatter(idx_v, [kv, pidx], cidx)
                    w_v[pl.ds(k * _N_PTS + b * _L, _L)] = wzs[kz] * wys[ky]

        descs = [pltpu.async_copy(tab.at[idx_v.at[k]], dst_v.at[k], sem)
                 for k in range(4)]
        for dsc in descs:
            dsc.wait()

        for b in range(_NB):
            pidx = b * _L + iota
            wx0 = w_v[pl.ds(4 * _N_PTS + b * _L, _L)]
            wx1 = w_v[pl.ds(5 * _N_PTS + b * _L, _L)]
            coff = co_v[pl.ds(b * _L, _L)]
            acc = [zero, zero, zero, zero]
            for k in range(4):
                kv = jnp.full((_L,), k, jnp.int32)
                wzy = w_v[pl.ds(k * _N_PTS + b * _L, _L)]
                wa = wzy * wx0
                wb = wzy * wx1
                for c in range(4):
                    cv = jnp.full((_L,), c, jnp.int32)
                    v0 = plsc.load_gather(dst_v, [kv, pidx, cv])
                    v1 = plsc.load_gather(dst_v, [kv, pidx, cv + coff])
                    acc[c] = acc[c] + wa * v0 + wb * v1
            dout_v[pl.ds(b * _L, _L)] = acc[0]
            cbase = pidx * 3
            plsc.store_scatter(cout_v, [cbase], acc[1])
            plsc.store_scatter(cout_v, [cbase + 1], acc[2])
            plsc.store_scatter(cout_v, [cbase + 2], acc[3])

        pltpu.sync_copy(dout_v, rawd.at[r0 + r])
        pltpu.sync_copy(cout_v, rawc.at[r0 + r])
        return carry

    lax.fori_loop(0, _RAYS_PER_W, ray_body, 0)


@jax.jit
def _sc_sample(origins_f, directions_f, lengths_f, density_f, color_f):
    mesh = plsc.VectorSubcoreMesh(core_axis_name="c", subcore_axis_name="s")
    fn = functools.partial(
        pl.kernel,
        out_type=(
            jax.ShapeDtypeStruct((_N_RAYS, _N_PTS), jnp.float32),
            jax.ShapeDtypeStruct((_N_RAYS, 3 * _N_PTS), jnp.float32),
            jax.ShapeDtypeStruct((_NV, 8), jnp.float32),
        ),
        mesh=mesh,
        compiler_params=pltpu.CompilerParams(
            needs_layout_passes=False, use_tc_tiling_on_sc=False),
        scratch_types=[
            pltpu.VMEM((_RAYS_PER_W * 3,), jnp.float32),
            pltpu.VMEM((_RAYS_PER_W * 3,), jnp.float32),
            pltpu.VMEM((_RAYS_PER_W * _N_PTS,), jnp.float32),
            pltpu.VMEM((4, _N_PTS), jnp.int32),
            pltpu.VMEM((6 * _N_PTS,), jnp.float32),
            pltpu.VMEM((_N_PTS,), jnp.int32),
            pltpu.VMEM((4, _N_PTS, 8), jnp.float32),
            pltpu.VMEM((_N_PTS,), jnp.float32),
            pltpu.VMEM((3 * _N_PTS,), jnp.float32),
            pltpu.VMEM((_CH_IN,), jnp.float32),
            pltpu.VMEM((_CH_IN,), jnp.float32),
            pltpu.VMEM((_CH_IN,), jnp.float32),
            pltpu.VMEM((_CH_IN,), jnp.float32),
            pltpu.VMEM((_CH_ROWS, 8), jnp.float32),
            pltpu.SemaphoreType.DMA,
        ],
    )(_sc_body)
    rawd, rawc, _ = fn(origins_f, directions_f, lengths_f, density_f, color_f)
    return rawd, rawc


def _act_body(dir_ref, rawd_ref, rawc_ref, dens_ref, col_ref):
    dvec = dir_ref[...]
    interval = jnp.sqrt(jnp.sum(dvec * dvec, axis=1, keepdims=True))
    a = rawd_ref[...] + _ACT_SHIFT
    dens_ref[...] = 1.0 - (1.0 + jnp.exp(a)) ** (-interval)
    col_ref[...] = jax.nn.sigmoid(rawc_ref[...])


@jax.jit
def _activations(directions, rawd, rawc):
    blk = 1024
    grid = (_N_RAYS // blk,)
    return pl.pallas_call(
        _act_body,
        grid=grid,
        in_specs=[
            pl.BlockSpec((blk, 3), lambda i: (i, 0)),
            pl.BlockSpec((blk, _N_PTS), lambda i: (i, 0)),
            pl.BlockSpec((blk, 3 * _N_PTS), lambda i: (i, 0)),
        ],
        out_specs=[
            pl.BlockSpec((blk, _N_PTS), lambda i: (i, 0)),
            pl.BlockSpec((blk, 3 * _N_PTS), lambda i: (i, 0)),
        ],
        out_shape=[
            jax.ShapeDtypeStruct((_N_RAYS, _N_PTS), jnp.float32),
            jax.ShapeDtypeStruct((_N_RAYS, 3 * _N_PTS), jnp.float32),
        ],
    )(directions, rawd, rawc)


def kernel(origins, directions, lengths, density_grid, color_grid):
    rawd, rawc = _sc_sample(
        origins.reshape(-1), directions.reshape(-1), lengths.reshape(-1),
        density_grid.reshape(-1), color_grid.reshape(-1))
    dens, col = _activations(directions, rawd, rawc)
    return (dens.reshape(_N_RAYS, _N_PTS, 1), col.reshape(_N_RAYS, _N_PTS, 3))
